# Initial kernel scaffold; baseline (speedup 1.0000x reference)
import jax
import jax.numpy as jnp
from jax.experimental import pallas as pl


def kernel(q, k, Wq, bq, Wk, bk, proj_n, proj_back_n):
    raise NotImplementedError("write your pallas kernel here")


# speedup vs baseline: 52.4431x; 52.4431x over previous
import jax
import jax.numpy as jnp
from jax.experimental import pallas as pl

_B, _H, _N, _HD = 1, 16, 2048, 64
_RC, _RN = 32, 256
_SCALE = 16 ** (-0.5)
_TOPK = 32
_BUDGET = 512
_BASIS_THR = 0.02
_BM = 256


def _kth_largest_thresh(x, kk):
    bits = jax.lax.bitcast_convert_type(x, jnp.int32)
    t = jnp.zeros((x.shape[0], 1), jnp.int32)
    for b in range(30, -1, -1):
        cand = t | (1 << b)
        cnt = jnp.sum((bits >= cand).astype(jnp.int32), axis=-1, keepdims=True)
        t = jnp.where(cnt >= kk, cand, t)
    return jax.lax.bitcast_convert_type(t, jnp.float32)


def _prep_kernel(q_ref, k_ref, wq_ref, bq_ref, wk_ref, bk_ref, pn_ref, pbn_ref,
                 qp_ref, kpr_ref, bt_ref):
    h = pl.program_id(0)
    qp_ref[0] = (jnp.dot(q_ref[0], wq_ref[...],
                         preferred_element_type=jnp.float32) + bq_ref[0])
    kp = (jnp.dot(k_ref[0], wk_ref[...],
                  preferred_element_type=jnp.float32) + bk_ref[0])
    kpr_ref[0] = jax.lax.dot_general(
        kp, pn_ref[...], (((0,), (0,)), ((), ())),
        preferred_element_type=jnp.float32)

    @pl.when(h == 0)
    def _():
        ab = jnp.abs(pbn_ref[...])
        bt_ref[...] = jnp.where(ab > _BASIS_THR, ab, 0.0)


def _main_kernel(qs_ref, kpr_ref, bt_ref, sp_ref, ap_ref, mk_ref):
    cheap = jnp.dot(qs_ref[0], kpr_ref[0],
                    preferred_element_type=jnp.float32) * _SCALE
    mx = jnp.max(cheap, axis=-1, keepdims=True)
    e = jnp.exp(cheap - mx)
    p = e / jnp.sum(e, axis=-1, keepdims=True)
    t32 = _kth_largest_thresh(p, _TOPK)
    sp = jnp.where(p >= t32, p, 0.0)
    sp_ref[0] = sp
    ap = jax.lax.dot_general(sp, bt_ref[...], (((1,), (1,)), ((), ())),
                             preferred_element_type=jnp.float32)
    ap_ref[0] = ap
    t512 = _kth_largest_thresh(ap, _BUDGET)
    mk_ref[0] = (ap >= t512).astype(jnp.float32)


def kernel(q, k, Wq, bq, Wk, bk, proj_n, proj_back_n):
    q2 = q.reshape(_H, _N, _HD)
    k2 = k.reshape(_H, _N, _HD)
    qp, kpr, bt = pl.pallas_call(
        _prep_kernel,
        grid=(_H,),
        in_specs=[
            pl.BlockSpec((1, _N, _HD), lambda h: (h, 0, 0)),
            pl.BlockSpec((1, _N, _HD), lambda h: (h, 0, 0)),
            pl.BlockSpec((_HD, _RC), lambda h: (0, 0)),
            pl.BlockSpec((1, _RC), lambda h: (0, 0)),
            pl.BlockSpec((_HD, _RC), lambda h: (0, 0)),
            pl.BlockSpec((1, _RC), lambda h: (0, 0)),
            pl.BlockSpec((_N, _RN), lambda h: (0, 0)),
            pl.BlockSpec((_N, _RN), lambda h: (0, 0)),
        ],
        out_specs=[
            pl.BlockSpec((1, _N, _RC), lambda h: (h, 0, 0)),
            pl.BlockSpec((1, _RC, _RN), lambda h: (h, 0, 0)),
            pl.BlockSpec((_N, _RN), lambda h: (0, 0)),
        ],
        out_shape=[
            jax.ShapeDtypeStruct((_H, _N, _RC), jnp.float32),
            jax.ShapeDtypeStruct((_H, _RC, _RN), jnp.float32),
            jax.ShapeDtypeStruct((_N, _RN), jnp.float32),
        ],
    )(q2, k2, Wq, bq.reshape(1, _RC), Wk, bk.reshape(1, _RC),
      proj_n, proj_back_n)

    qs = qp[:, 1:, :]
    nm = (_N - 1 + _BM - 1) // _BM
    sp, ap, mk = pl.pallas_call(
        _main_kernel,
        grid=(_H, nm),
        in_specs=[
            pl.BlockSpec((1, _BM, _RC), lambda h, m: (h, m, 0)),
            pl.BlockSpec((1, _RC, _RN), lambda h, m: (h, 0, 0)),
            pl.BlockSpec((_N, _RN), lambda h, m: (0, 0)),
        ],
        out_specs=[
            pl.BlockSpec((1, _BM, _RN), lambda h, m: (h, m, 0)),
            pl.BlockSpec((1, _BM, _N), lambda h, m: (h, m, 0)),
            pl.BlockSpec((1, _BM, _N), lambda h, m: (h, m, 0)),
        ],
        out_shape=[
            jax.ShapeDtypeStruct((_H, _N - 1, _RN), jnp.float32),
            jax.ShapeDtypeStruct((_H, _N - 1, _N), jnp.float32),
            jax.ShapeDtypeStruct((_H, _N - 1, _N), jnp.float32),
        ],
    )(qs, kpr, bt)

    cls = jnp.ones((_H, 1, _N), jnp.float32)
    attn_mask = jnp.concatenate([cls, mk], axis=1)
    return (sp.reshape(_B, _H, _N - 1, _RN),
            ap.reshape(_B, _H, _N - 1, _N),
            attn_mask.reshape(_B, _H, _N, _N))


# speedup vs baseline: 56.9535x; 1.0860x over previous
import jax
import jax.numpy as jnp
from jax.experimental import pallas as pl
from jax.experimental.pallas import tpu as pltpu

_B, _H, _N, _HD = 1, 16, 2048, 64
_RC, _RN = 32, 256
_SCALE = 16 ** (-0.5)
_TOPK = 32
_BUDGET = 512
_BASIS_THR = 0.02
_BM = 256


def _kth_largest_thresh(x, kk):
    bits = jax.lax.bitcast_convert_type(x, jnp.int32)
    t = jnp.zeros((x.shape[0], 1), jnp.int32)
    for b in range(30, -1, -1):
        cand = t | (1 << b)
        cnt = jnp.sum((bits >= cand).astype(jnp.int32), axis=-1, keepdims=True)
        t = jnp.where(cnt >= kk, cand, t)
    return jax.lax.bitcast_convert_type(t, jnp.float32)


def _prep_kernel(q_ref, k_ref, wq_ref, bq_ref, wk_ref, bk_ref, pn_ref, pbn_ref,
                 qs_ref, kpr_ref, bt_ref):
    h = pl.program_id(0)
    qp = (jnp.dot(q_ref[0], wq_ref[...],
                  preferred_element_type=jnp.float32) + bq_ref[0])
    qs_ref[0] = qp[1:, :]
    kp = (jnp.dot(k_ref[0], wk_ref[...],
                  preferred_element_type=jnp.float32) + bk_ref[0])
    kpr_ref[0] = jax.lax.dot_general(
        kp, pn_ref[...], (((0,), (0,)), ((), ())),
        preferred_element_type=jnp.float32)

    @pl.when(h == 0)
    def _():
        ab = jnp.abs(pbn_ref[...])
        bt_ref[...] = jnp.where(ab > _BASIS_THR, ab, 0.0)


def _main_kernel(qs_ref, kpr_ref, bt_ref, sp_ref, ap_ref, mk_ref, carry_ref):
    m = pl.program_id(1)
    cheap = jnp.dot(qs_ref[0], kpr_ref[0],
                    preferred_element_type=jnp.float32) * _SCALE
    mx = jnp.max(cheap, axis=-1, keepdims=True)
    e = jnp.exp(cheap - mx)
    p = e / jnp.sum(e, axis=-1, keepdims=True)
    t32 = _kth_largest_thresh(p, _TOPK)
    sp = jnp.where(p >= t32, p, 0.0)
    sp_ref[0] = sp
    ap = jax.lax.dot_general(sp, bt_ref[...], (((1,), (1,)), ((), ())),
                             preferred_element_type=jnp.float32)
    ap_ref[0] = ap
    t512 = _kth_largest_thresh(ap, _BUDGET)
    mk = (ap >= t512).astype(jnp.float32)
    row0 = jnp.where(m == 0, jnp.ones((1, _N), jnp.float32), carry_ref[...])
    mk_ref[0, 0:1, :] = row0
    mk_ref[0, 1:_BM, :] = mk[:_BM - 1, :]
    carry_ref[...] = mk[_BM - 1:, :]


def kernel(q, k, Wq, bq, Wk, bk, proj_n, proj_back_n):
    q2 = q.reshape(_H, _N, _HD)
    k2 = k.reshape(_H, _N, _HD)
    qs, kpr, bt = pl.pallas_call(
        _prep_kernel,
        grid=(_H,),
        in_specs=[
            pl.BlockSpec((1, _N, _HD), lambda h: (h, 0, 0)),
            pl.BlockSpec((1, _N, _HD), lambda h: (h, 0, 0)),
            pl.BlockSpec((_HD, _RC), lambda h: (0, 0)),
            pl.BlockSpec((1, _RC), lambda h: (0, 0)),
            pl.BlockSpec((_HD, _RC), lambda h: (0, 0)),
            pl.BlockSpec((1, _RC), lambda h: (0, 0)),
            pl.BlockSpec((_N, _RN), lambda h: (0, 0)),
            pl.BlockSpec((_N, _RN), lambda h: (0, 0)),
        ],
        out_specs=[
            pl.BlockSpec((1, _N - 1, _RC), lambda h: (h, 0, 0)),
            pl.BlockSpec((1, _RC, _RN), lambda h: (h, 0, 0)),
            pl.BlockSpec((_N, _RN), lambda h: (0, 0)),
        ],
        out_shape=[
            jax.ShapeDtypeStruct((_H, _N - 1, _RC), jnp.float32),
            jax.ShapeDtypeStruct((_H, _RC, _RN), jnp.float32),
            jax.ShapeDtypeStruct((_N, _RN), jnp.float32),
        ],
    )(q2, k2, Wq, bq.reshape(1, _RC), Wk, bk.reshape(1, _RC),
      proj_n, proj_back_n)

    nm = (_N - 1 + _BM - 1) // _BM
    sp, ap, mk = pl.pallas_call(
        _main_kernel,
        grid=(_H, nm),
        in_specs=[
            pl.BlockSpec((1, _BM, _RC), lambda h, m: (h, m, 0)),
            pl.BlockSpec((1, _RC, _RN), lambda h, m: (h, 0, 0)),
            pl.BlockSpec((_N, _RN), lambda h, m: (0, 0)),
        ],
        out_specs=[
            pl.BlockSpec((1, _BM, _RN), lambda h, m: (h, m, 0)),
            pl.BlockSpec((1, _BM, _N), lambda h, m: (h, m, 0)),
            pl.BlockSpec((1, _BM, _N), lambda h, m: (h, m, 0)),
        ],
        out_shape=[
            jax.ShapeDtypeStruct((_H, _N - 1, _RN), jnp.float32),
            jax.ShapeDtypeStruct((_H, _N - 1, _N), jnp.float32),
            jax.ShapeDtypeStruct((_H, _N, _N), jnp.float32),
        ],
        scratch_shapes=[pltpu.VMEM((1, _N), jnp.float32)],
    )(qs, kpr, bt)

    return (sp.reshape(_B, _H, _N - 1, _RN),
            ap.reshape(_B, _H, _N - 1, _N),
            mk.reshape(_B, _H, _N, _N))
